# baseline (speedup 1.0000x reference)
import functools

import jax
import jax.numpy as jnp
from jax import lax
from jax.experimental import pallas as pl
from jax.experimental.pallas import tpu as pltpu
from jax.experimental.pallas import tpu_sc as plsc

N = 10000
E = 160000
D_IN = 128
D_EDGE = 4
D_H = 32

NC = 2
NS = 16
NW = NC * NS
PER_W = E // NW
CHUNK = 64
NFULL = PER_W // CHUNK
TAIL = PER_W - NFULL * CHUNK
NPAD = 10240
ROWS_PER_TILE = NPAD // NS
L = 16


@functools.lru_cache(maxsize=None)
def _sc_kernels():
  mesh = plsc.VectorSubcoreMesh(core_axis_name="c", subcore_axis_name="s",
                                num_cores=NC, num_subcores=NS)

  @functools.partial(
    pl.kernel,
    out_type=jax.ShapeDtypeStruct((NC, NPAD, D_IN), jnp.float32),
    mesh=mesh,
    scratch_types=[
        pltpu.VMEM((CHUNK,), jnp.int32),
        pltpu.VMEM((CHUNK,), jnp.int32),
        pltpu.VMEM((CHUNK,), jnp.int32),
        pltpu.VMEM((CHUNK,), jnp.int32),
        pltpu.VMEM((CHUNK * D_EDGE + L,), jnp.float32),
        pltpu.VMEM((CHUNK * D_EDGE + L,), jnp.float32),
        pltpu.VMEM((CHUNK, D_IN), jnp.float32),
        pltpu.VMEM((CHUNK, D_IN), jnp.float32),
        pltpu.VMEM((CHUNK, D_IN), jnp.float32),
        pltpu.VMEM((CHUNK, D_IN), jnp.float32),
        pltpu.VMEM_SHARED((NPAD, D_IN), jnp.float32),
        pltpu.SemaphoreType.DMA,
        pltpu.SemaphoreType.DMA,
    ],
  )
  def _edge_fused(y_hbm, col_hbm, row_hbm, ea_hbm, zeros_hbm, out_hbm,
                  cidx0, cidx1, ridx0, ridx1, ea0, ea1, yg0, yg1, m0, m1,
                  acc_sh, sem0, sem1):
    cid = lax.axis_index("c")
    sid = lax.axis_index("s")
    wid = sid * NC + cid
    base = wid * PER_W
    stripe = pl.ds(sid * ROWS_PER_TILE, ROWS_PER_TILE)

    bufs = ((cidx0, ridx0, ea0, yg0, m0, sem0),
            (cidx1, ridx1, ea1, yg1, m1, sem1))

    pltpu.sync_copy(zeros_hbm.at[stripe], acc_sh.at[stripe])

    zed = jnp.zeros((L,), jnp.float32)

    def zrow(r, carry):
        for sj in range(D_IN // L):
            m0[r, pl.ds(sj * L, L)] = zed
            m1[r, pl.ds(sj * L, L)] = zed
        return carry

    lax.fori_loop(0, CHUNK, zrow, 0)
    plsc.subcore_barrier()

    def edge_math(ea_v, yg_v, m_v):
        def body(e):
            av = ea_v[pl.ds(D_EDGE * e, L)]
            a0 = av[0]
            a1 = av[1]
            a2 = av[2]
            a3 = av[3]
            for q in range(D_H // L):
                v = (a0 * yg_v[e, pl.ds(q * L, L)]
                     + a1 * yg_v[e, pl.ds(32 + q * L, L)]
                     + a2 * yg_v[e, pl.ds(64 + q * L, L)]
                     + a3 * yg_v[e, pl.ds(96 + q * L, L)])
                s = 1.0 / (1.0 + jnp.exp(-v))
                m_v[e, pl.ds(q * L, L)] = v * s
        return body

    def issue(i, b):
        cidx, ridx, ea, yg, m, sem = bufs[b]
        off = base + i * CHUNK
        pltpu.sync_copy(col_hbm.at[pl.ds(off, CHUNK)], cidx)
        pltpu.sync_copy(row_hbm.at[pl.ds(off, CHUNK)], ridx)
        pltpu.sync_copy(ea_hbm.at[pl.ds(off * D_EDGE, CHUNK * D_EDGE)],
                        ea.at[pl.ds(0, CHUNK * D_EDGE)])
        return pltpu.async_copy(y_hbm.at[cidx], yg, sem)

    def crunch(b):
        cidx, ridx, ea, yg, m, sem = bufs[b]
        plsc.parallel_loop(0, CHUNK, 1, unroll=4)(edge_math(ea, yg, m))
        pltpu.sync_copy(m, acc_sh.at[ridx], add=True)

    issue(0, 0).wait()

    def pair_body(p, carry):
        d1 = issue(2 * p + 1, 1)
        crunch(0)
        d1.wait()
        d0 = issue(2 * p + 2, 0)
        crunch(1)
        d0.wait()
        return carry

    lax.fori_loop(0, NFULL // 2 - 1, pair_body, 0)
    dl = issue(NFULL - 1, 1)
    crunch(0)
    dl.wait()
    crunch(1)

    off = base + NFULL * CHUNK
    pltpu.sync_copy(col_hbm.at[pl.ds(off, TAIL)], cidx1.at[pl.ds(0, TAIL)])
    pltpu.sync_copy(row_hbm.at[pl.ds(off, TAIL)], ridx1.at[pl.ds(0, TAIL)])
    pltpu.sync_copy(ea_hbm.at[pl.ds(off * D_EDGE, TAIL * D_EDGE)],
                    ea1.at[pl.ds(0, TAIL * D_EDGE)])
    pltpu.async_copy(y_hbm.at[cidx1.at[pl.ds(0, TAIL)]],
                     yg1.at[pl.ds(0, TAIL)], sem1).wait()
    plsc.parallel_loop(0, TAIL, 1, unroll=4)(edge_math(ea1, yg1, m1))
    pltpu.sync_copy(m1.at[pl.ds(0, TAIL)],
                    acc_sh.at[ridx1.at[pl.ds(0, TAIL)]], add=True)

    plsc.subcore_barrier()
    pltpu.sync_copy(acc_sh.at[stripe], out_hbm.at[cid, stripe])

  return _edge_fused


BY = 2000


def _prep_body(x_ref, w1_ref, y_ref):
    y_ref[...] = jnp.dot(x_ref[...], w1_ref[...],
                         preferred_element_type=jnp.float32)


def _prep_stage(x, w1):
    return pl.pallas_call(
        _prep_body,
        grid=(N // BY,),
        in_specs=[
            pl.BlockSpec((BY, D_IN), lambda i: (i, 0)),
            pl.BlockSpec((D_IN, D_EDGE * D_H), lambda i: (0, 0)),
        ],
        out_specs=pl.BlockSpec((BY, D_EDGE * D_H), lambda i: (i, 0)),
        out_shape=jax.ShapeDtypeStruct((N, D_EDGE * D_H), jnp.float32),
    )(x, w1)


BN = 1000


def _update_body(x_ref, p0_ref, p1_ref, wl_ref, w2_ref, wl2_ref, out_ref):
    x = x_ref[...]
    agg = p0_ref[:, :D_H] + p1_ref[:, :D_H]
    agg = jnp.dot(agg, wl_ref[...], preferred_element_type=jnp.float32)
    t = jnp.dot(x, w2_ref[...], preferred_element_type=jnp.float32)
    u = agg[:, 0:1] * t[:, 0:32]
    for h in range(1, D_H):
        u += agg[:, h:h + 1] * t[:, h * 32:(h + 1) * 32]
    u = u * jax.nn.sigmoid(u)
    out_ref[...] = x + jnp.dot(u, wl2_ref[...], preferred_element_type=jnp.float32)


def _update_stage(x, p0, p1, wl, w2, wl2):
    return pl.pallas_call(
        _update_body,
        grid=(N // BN,),
        in_specs=[
            pl.BlockSpec((BN, D_IN), lambda i: (i, 0)),
            pl.BlockSpec((BN, D_IN), lambda i: (i, 0)),
            pl.BlockSpec((BN, D_IN), lambda i: (i, 0)),
            pl.BlockSpec((D_H, D_H), lambda i: (0, 0)),
            pl.BlockSpec((D_IN, D_H * D_H), lambda i: (0, 0)),
            pl.BlockSpec((D_H, D_IN), lambda i: (0, 0)),
        ],
        out_specs=pl.BlockSpec((BN, D_IN), lambda i: (i, 0)),
        out_shape=jax.ShapeDtypeStruct((N, D_IN), jnp.float32),
    )(x, p0, p1, wl, w2, wl2)


def kernel(node_features, edge_index, edge_attr_e3nn, node_attr_scalar_raw,
           W_tp_msg, W_lin_msg, W_tp_upd, W_lin_upd):
    del node_attr_scalar_raw
    row = edge_index[0].astype(jnp.int32)
    col = edge_index[1].astype(jnp.int32)
    w1 = W_tp_msg.reshape(D_IN, D_EDGE * D_H)
    w2 = W_tp_upd.reshape(D_IN, D_H * D_H)
    ea1d = edge_attr_e3nn.reshape(-1)
    zeros = jnp.zeros((NPAD, D_IN), jnp.float32)

    y = _prep_stage(node_features, w1)
    _edge_fused = _sc_kernels()
    partials = _edge_fused(y, col, row, ea1d, zeros)
    return _update_stage(node_features, partials[0, :N], partials[1, :N],
                         W_lin_msg, w2, W_lin_upd)


# speedup vs baseline: 1.1404x; 1.1404x over previous
import functools

import jax
import jax.numpy as jnp
from jax import lax
from jax.experimental import pallas as pl
from jax.experimental.pallas import tpu as pltpu
from jax.experimental.pallas import tpu_sc as plsc

N = 10000
E = 160000
D_IN = 128
D_EDGE = 4
D_H = 32

NC = 2
NS = 16
NW = NC * NS
PER_W = E // NW
CHUNK = 104
NFULL = PER_W // CHUNK
TAIL = PER_W - NFULL * CHUNK
NPAD = 10240
ROWS_PER_TILE = NPAD // NS
L = 16


@functools.lru_cache(maxsize=None)
def _sc_kernels():
  mesh = plsc.VectorSubcoreMesh(core_axis_name="c", subcore_axis_name="s",
                                num_cores=NC, num_subcores=NS)

  @functools.partial(
    pl.kernel,
    out_type=jax.ShapeDtypeStruct((NC, NPAD, D_IN), jnp.float32),
    mesh=mesh,
    scratch_types=[
        pltpu.VMEM((CHUNK,), jnp.int32),
        pltpu.VMEM((CHUNK,), jnp.int32),
        pltpu.VMEM((CHUNK,), jnp.int32),
        pltpu.VMEM((CHUNK,), jnp.int32),
        pltpu.VMEM((CHUNK * D_EDGE + L,), jnp.float32),
        pltpu.VMEM((CHUNK * D_EDGE + L,), jnp.float32),
        pltpu.VMEM((CHUNK, D_IN), jnp.float32),
        pltpu.VMEM((CHUNK, D_IN), jnp.float32),
        pltpu.VMEM((CHUNK, D_IN), jnp.float32),
        pltpu.VMEM_SHARED((NPAD, D_IN), jnp.float32),
        pltpu.SemaphoreType.DMA,
        pltpu.SemaphoreType.DMA,
    ],
  )
  def _edge_fused(y_hbm, col_hbm, row_hbm, ea_hbm, zeros_hbm, out_hbm,
                  cidx0, cidx1, ridx0, ridx1, ea0, ea1, yg0, yg1, m0,
                  acc_sh, sem0, sem1):
    m1 = m0
    cid = lax.axis_index("c")
    sid = lax.axis_index("s")
    wid = sid * NC + cid
    base = wid * PER_W
    stripe = pl.ds(sid * ROWS_PER_TILE, ROWS_PER_TILE)

    bufs = ((cidx0, ridx0, ea0, yg0, m0, sem0),
            (cidx1, ridx1, ea1, yg1, m1, sem1))

    pltpu.sync_copy(zeros_hbm.at[stripe], acc_sh.at[stripe])

    zed = jnp.zeros((L,), jnp.float32)

    def zrow(r, carry):
        for sj in range(D_IN // L):
            m0[r, pl.ds(sj * L, L)] = zed
        return carry

    lax.fori_loop(0, CHUNK, zrow, 0)
    plsc.subcore_barrier()

    def edge_math(ea_v, yg_v, m_v):
        def body(e):
            av = ea_v[pl.ds(D_EDGE * e, L)]
            a0 = av[0]
            a1 = av[1]
            a2 = av[2]
            a3 = av[3]
            for q in range(D_H // L):
                v = (a0 * yg_v[e, pl.ds(q * L, L)]
                     + a1 * yg_v[e, pl.ds(32 + q * L, L)]
                     + a2 * yg_v[e, pl.ds(64 + q * L, L)]
                     + a3 * yg_v[e, pl.ds(96 + q * L, L)])
                s = 1.0 / (1.0 + jnp.exp(-v))
                m_v[e, pl.ds(q * L, L)] = v * s
        return body

    def issue(i, b):
        cidx, ridx, ea, yg, m, sem = bufs[b]
        off = base + i * CHUNK
        pltpu.sync_copy(col_hbm.at[pl.ds(off, CHUNK)], cidx)
        pltpu.sync_copy(row_hbm.at[pl.ds(off, CHUNK)], ridx)
        pltpu.sync_copy(ea_hbm.at[pl.ds(off * D_EDGE, CHUNK * D_EDGE)],
                        ea.at[pl.ds(0, CHUNK * D_EDGE)])
        return pltpu.async_copy(y_hbm.at[cidx], yg, sem)

    def crunch(b):
        cidx, ridx, ea, yg, m, sem = bufs[b]
        plsc.parallel_loop(0, CHUNK, 1, unroll=4)(edge_math(ea, yg, m))
        pltpu.sync_copy(m, acc_sh.at[ridx], add=True)

    issue(0, 0).wait()

    def pair_body(p, carry):
        d1 = issue(2 * p + 1, 1)
        crunch(0)
        d1.wait()
        d0 = issue(2 * p + 2, 0)
        crunch(1)
        d0.wait()
        return carry

    lax.fori_loop(0, NFULL // 2 - 1, pair_body, 0)
    dl = issue(NFULL - 1, 1)
    crunch(0)
    dl.wait()
    crunch(1)

    off = base + NFULL * CHUNK
    pltpu.sync_copy(col_hbm.at[pl.ds(off, TAIL)], cidx1.at[pl.ds(0, TAIL)])
    pltpu.sync_copy(row_hbm.at[pl.ds(off, TAIL)], ridx1.at[pl.ds(0, TAIL)])
    pltpu.sync_copy(ea_hbm.at[pl.ds(off * D_EDGE, TAIL * D_EDGE)],
                    ea1.at[pl.ds(0, TAIL * D_EDGE)])
    pltpu.async_copy(y_hbm.at[cidx1.at[pl.ds(0, TAIL)]],
                     yg1.at[pl.ds(0, TAIL)], sem1).wait()
    plsc.parallel_loop(0, TAIL, 1, unroll=4)(edge_math(ea1, yg1, m1))
    pltpu.sync_copy(m1.at[pl.ds(0, TAIL)],
                    acc_sh.at[ridx1.at[pl.ds(0, TAIL)]], add=True)

    plsc.subcore_barrier()
    pltpu.sync_copy(acc_sh.at[stripe], out_hbm.at[cid, stripe])

  return _edge_fused


BY = 2000


def _prep_body(x_ref, w1_ref, y_ref):
    y_ref[...] = jnp.dot(x_ref[...], w1_ref[...],
                         preferred_element_type=jnp.float32)


def _prep_stage(x, w1):
    return pl.pallas_call(
        _prep_body,
        grid=(N // BY,),
        in_specs=[
            pl.BlockSpec((BY, D_IN), lambda i: (i, 0)),
            pl.BlockSpec((D_IN, D_EDGE * D_H), lambda i: (0, 0)),
        ],
        out_specs=pl.BlockSpec((BY, D_EDGE * D_H), lambda i: (i, 0)),
        out_shape=jax.ShapeDtypeStruct((N, D_EDGE * D_H), jnp.float32),
    )(x, w1)


BN = 1000


def _update_body(x_ref, p0_ref, p1_ref, wl_ref, w2_ref, wl2_ref, out_ref):
    x = x_ref[...]
    agg = p0_ref[:, :D_H] + p1_ref[:, :D_H]
    agg = jnp.dot(agg, wl_ref[...], preferred_element_type=jnp.float32)
    t = jnp.dot(x, w2_ref[...], preferred_element_type=jnp.float32)
    u = agg[:, 0:1] * t[:, 0:32]
    for h in range(1, D_H):
        u += agg[:, h:h + 1] * t[:, h * 32:(h + 1) * 32]
    u = u * jax.nn.sigmoid(u)
    out_ref[...] = x + jnp.dot(u, wl2_ref[...], preferred_element_type=jnp.float32)


def _update_stage(x, p0, p1, wl, w2, wl2):
    return pl.pallas_call(
        _update_body,
        grid=(N // BN,),
        in_specs=[
            pl.BlockSpec((BN, D_IN), lambda i: (i, 0)),
            pl.BlockSpec((BN, D_IN), lambda i: (i, 0)),
            pl.BlockSpec((BN, D_IN), lambda i: (i, 0)),
            pl.BlockSpec((D_H, D_H), lambda i: (0, 0)),
            pl.BlockSpec((D_IN, D_H * D_H), lambda i: (0, 0)),
            pl.BlockSpec((D_H, D_IN), lambda i: (0, 0)),
        ],
        out_specs=pl.BlockSpec((BN, D_IN), lambda i: (i, 0)),
        out_shape=jax.ShapeDtypeStruct((N, D_IN), jnp.float32),
    )(x, p0, p1, wl, w2, wl2)


def kernel(node_features, edge_index, edge_attr_e3nn, node_attr_scalar_raw,
           W_tp_msg, W_lin_msg, W_tp_upd, W_lin_upd):
    del node_attr_scalar_raw
    row = edge_index[0].astype(jnp.int32)
    col = edge_index[1].astype(jnp.int32)
    w1 = W_tp_msg.reshape(D_IN, D_EDGE * D_H)
    w2 = W_tp_upd.reshape(D_IN, D_H * D_H)
    ea1d = edge_attr_e3nn.reshape(-1)
    zeros = jnp.zeros((NPAD, D_IN), jnp.float32)

    y = _prep_stage(node_features, w1)
    _edge_fused = _sc_kernels()
    partials = _edge_fused(y, col, row, ea1d, zeros)
    return _update_stage(node_features, partials[0, :N], partials[1, :N],
                         W_lin_msg, w2, W_lin_upd)


# speedup vs baseline: 1.3696x; 1.2010x over previous
import functools

import jax
import jax.numpy as jnp
from jax import lax
from jax.experimental import pallas as pl
from jax.experimental.pallas import tpu as pltpu
from jax.experimental.pallas import tpu_sc as plsc

N = 10000
E = 160000
D_IN = 128
D_EDGE = 4
D_H = 32

NC = 2
NS = 16
NW = NC * NS
PER_W = E // NW
CHUNK = 104
NFULL = PER_W // CHUNK
TAIL = PER_W - NFULL * CHUNK
NPAD = 10240
ROWS_PER_TILE = NPAD // NS
L = 16


@functools.lru_cache(maxsize=None)
def _sc_kernels():
  mesh = plsc.VectorSubcoreMesh(core_axis_name="c", subcore_axis_name="s",
                                num_cores=NC, num_subcores=NS)

  @functools.partial(
    pl.kernel,
    out_type=jax.ShapeDtypeStruct((NC, NPAD, D_IN), jnp.float32),
    mesh=mesh,
    scratch_types=[
        pltpu.VMEM((CHUNK,), jnp.int32),
        pltpu.VMEM((CHUNK,), jnp.int32),
        pltpu.VMEM((CHUNK,), jnp.int32),
        pltpu.VMEM((CHUNK,), jnp.int32),
        pltpu.VMEM((CHUNK * D_EDGE + L,), jnp.float32),
        pltpu.VMEM((CHUNK * D_EDGE + L,), jnp.float32),
        pltpu.VMEM((CHUNK, D_IN), jnp.float32),
        pltpu.VMEM((CHUNK, D_IN), jnp.float32),
        pltpu.VMEM((CHUNK, D_IN), jnp.float32),
        pltpu.VMEM_SHARED((NPAD, D_IN), jnp.float32),
        pltpu.SemaphoreType.DMA,
        pltpu.SemaphoreType.DMA,
    ],
  )
  def _edge_fused(y_hbm, col_hbm, row_hbm, ea_hbm, zeros_hbm, out_hbm,
                  cidx0, cidx1, ridx0, ridx1, ea0, ea1, yg0, yg1, m0,
                  acc_sh, sem0, sem1):
    m1 = m0
    cid = lax.axis_index("c")
    sid = lax.axis_index("s")
    wid = sid * NC + cid
    base = wid * PER_W
    stripe = pl.ds(sid * ROWS_PER_TILE, ROWS_PER_TILE)

    bufs = ((cidx0, ridx0, ea0, yg0, m0, sem0),
            (cidx1, ridx1, ea1, yg1, m1, sem1))

    pltpu.sync_copy(zeros_hbm.at[stripe], acc_sh.at[stripe])

    zed = jnp.zeros((L,), jnp.float32)

    def zrow(r, carry):
        for sj in range(D_IN // L):
            m0[r, pl.ds(sj * L, L)] = zed
        return carry

    lax.fori_loop(0, CHUNK, zrow, 0)
    plsc.subcore_barrier()

    def edge_math(ea_v, yg_v, m_v):
        def body(e):
            av = ea_v[pl.ds(D_EDGE * e, L)]
            a0 = av[0]
            a1 = av[1]
            a2 = av[2]
            a3 = av[3]
            for q in range(D_H // L):
                v = (a0 * yg_v[e, pl.ds(q * L, L)]
                     + a1 * yg_v[e, pl.ds(32 + q * L, L)]
                     + a2 * yg_v[e, pl.ds(64 + q * L, L)]
                     + a3 * yg_v[e, pl.ds(96 + q * L, L)])
                s = 1.0 / (1.0 + jnp.exp(-v))
                m_v[e, pl.ds(q * L, L)] = v * s
        return body

    def issue(i, b):
        cidx, ridx, ea, yg, m, sem = bufs[b]
        off = base + i * CHUNK
        pltpu.sync_copy(col_hbm.at[pl.ds(off, CHUNK)], cidx)
        pltpu.sync_copy(row_hbm.at[pl.ds(off, CHUNK)], ridx)
        pltpu.sync_copy(ea_hbm.at[pl.ds(off * D_EDGE, CHUNK * D_EDGE)],
                        ea.at[pl.ds(0, CHUNK * D_EDGE)])
        return pltpu.async_copy(y_hbm.at[cidx], yg, sem)

    def crunch(b):
        cidx, ridx, ea, yg, m, sem = bufs[b]
        plsc.parallel_loop(0, CHUNK, 1, unroll=4)(edge_math(ea, yg, m))
        pltpu.sync_copy(m, acc_sh.at[ridx], add=True)

    issue(0, 0).wait()

    def pair_body(p, carry):
        d1 = issue(2 * p + 1, 1)
        crunch(0)
        d1.wait()
        d0 = issue(2 * p + 2, 0)
        crunch(1)
        d0.wait()
        return carry

    lax.fori_loop(0, NFULL // 2 - 1, pair_body, 0)
    dl = issue(NFULL - 1, 1)
    crunch(0)
    dl.wait()
    crunch(1)

    off = base + NFULL * CHUNK
    pltpu.sync_copy(col_hbm.at[pl.ds(off, TAIL)], cidx1.at[pl.ds(0, TAIL)])
    pltpu.sync_copy(row_hbm.at[pl.ds(off, TAIL)], ridx1.at[pl.ds(0, TAIL)])
    pltpu.sync_copy(ea_hbm.at[pl.ds(off * D_EDGE, TAIL * D_EDGE)],
                    ea1.at[pl.ds(0, TAIL * D_EDGE)])
    pltpu.async_copy(y_hbm.at[cidx1.at[pl.ds(0, TAIL)]],
                     yg1.at[pl.ds(0, TAIL)], sem1).wait()
    plsc.parallel_loop(0, TAIL, 1, unroll=4)(edge_math(ea1, yg1, m1))
    pltpu.sync_copy(m1.at[pl.ds(0, TAIL)],
                    acc_sh.at[ridx1.at[pl.ds(0, TAIL)]], add=True)

    plsc.subcore_barrier()
    pltpu.sync_copy(acc_sh.at[stripe], out_hbm.at[cid, stripe])

  return _edge_fused


BY = 2000


def _prep_body(x_ref, w1_ref, y_ref):
    y_ref[...] = jnp.dot(x_ref[...], w1_ref[...],
                         preferred_element_type=jnp.float32)


def _prep_stage(x, w1):
    return pl.pallas_call(
        _prep_body,
        grid=(N // BY,),
        in_specs=[
            pl.BlockSpec((BY, D_IN), lambda i: (i, 0)),
            pl.BlockSpec((D_IN, D_EDGE * D_H), lambda i: (0, 0)),
        ],
        out_specs=pl.BlockSpec((BY, D_EDGE * D_H), lambda i: (i, 0)),
        out_shape=jax.ShapeDtypeStruct((N, D_EDGE * D_H), jnp.float32),
    )(x, w1)


BN = 1000


def _update_body(x_ref, p0_ref, p1_ref, wl_ref, w2_ref, sel_ref, wl2_ref, out_ref):
    x = x_ref[...]
    agg = p0_ref[:, :D_H] + p1_ref[:, :D_H]
    agg = jnp.dot(agg, wl_ref[...], preferred_element_type=jnp.float32)
    t = jnp.dot(x, w2_ref[...], preferred_element_type=jnp.float32)
    a32 = jnp.tile(agg, (1, D_H))
    u = jnp.dot(t * a32, sel_ref[...], preferred_element_type=jnp.float32)
    u = u * jax.nn.sigmoid(u)
    out_ref[...] = x + jnp.dot(u, wl2_ref[...], preferred_element_type=jnp.float32)


def _update_stage(x, p0, p1, wl, w2, sel, wl2):
    return pl.pallas_call(
        _update_body,
        grid=(N // BN,),
        in_specs=[
            pl.BlockSpec((BN, D_IN), lambda i: (i, 0)),
            pl.BlockSpec((BN, D_IN), lambda i: (i, 0)),
            pl.BlockSpec((BN, D_IN), lambda i: (i, 0)),
            pl.BlockSpec((D_H, D_H), lambda i: (0, 0)),
            pl.BlockSpec((D_IN, D_H * D_H), lambda i: (0, 0)),
            pl.BlockSpec((D_H * D_H, D_H), lambda i: (0, 0)),
            pl.BlockSpec((D_H, D_IN), lambda i: (0, 0)),
        ],
        out_specs=pl.BlockSpec((BN, D_IN), lambda i: (i, 0)),
        out_shape=jax.ShapeDtypeStruct((N, D_IN), jnp.float32),
    )(x, p0, p1, wl, w2, sel, wl2)


def kernel(node_features, edge_index, edge_attr_e3nn, node_attr_scalar_raw,
           W_tp_msg, W_lin_msg, W_tp_upd, W_lin_upd):
    del node_attr_scalar_raw
    row = edge_index[0].astype(jnp.int32)
    col = edge_index[1].astype(jnp.int32)
    w1 = W_tp_msg.reshape(D_IN, D_EDGE * D_H)
    w2 = W_tp_upd.transpose(0, 2, 1).reshape(D_IN, D_H * D_H)
    sel = jnp.kron(jnp.eye(D_H, dtype=jnp.float32),
                   jnp.ones((D_H, 1), jnp.float32))
    ea1d = edge_attr_e3nn.reshape(-1)
    zeros = jnp.zeros((NPAD, D_IN), jnp.float32)

    y = _prep_stage(node_features, w1)
    _edge_fused = _sc_kernels()
    partials = _edge_fused(y, col, row, ea1d, zeros)
    return _update_stage(node_features, partials[0, :N], partials[1, :N],
                         W_lin_msg, w2, sel, W_lin_upd)


# speedup vs baseline: 1.3717x; 1.0015x over previous
import functools

import jax
import jax.numpy as jnp
from jax import lax
from jax.experimental import pallas as pl
from jax.experimental.pallas import tpu as pltpu
from jax.experimental.pallas import tpu_sc as plsc

N = 10000
E = 160000
D_IN = 128
D_EDGE = 4
D_H = 32

NC = 2
NS = 16
NW = NC * NS
PER_W = E // NW
CHUNK = 104
NFULL = PER_W // CHUNK
TAIL = PER_W - NFULL * CHUNK
NPAD = 10240
ROWS_PER_TILE = NPAD // NS
L = 16


@functools.lru_cache(maxsize=None)
def _sc_kernels():
  mesh = plsc.VectorSubcoreMesh(core_axis_name="c", subcore_axis_name="s",
                                num_cores=NC, num_subcores=NS)

  @functools.partial(
    pl.kernel,
    out_type=jax.ShapeDtypeStruct((NC, NPAD, D_IN), jnp.float32),
    mesh=mesh,
    scratch_types=[
        pltpu.VMEM((CHUNK,), jnp.int32),
        pltpu.VMEM((CHUNK,), jnp.int32),
        pltpu.VMEM((CHUNK,), jnp.int32),
        pltpu.VMEM((CHUNK,), jnp.int32),
        pltpu.VMEM((CHUNK * D_EDGE + L,), jnp.float32),
        pltpu.VMEM((CHUNK * D_EDGE + L,), jnp.float32),
        pltpu.VMEM((CHUNK, D_IN), jnp.float32),
        pltpu.VMEM((CHUNK, D_IN), jnp.float32),
        pltpu.VMEM((CHUNK, D_IN), jnp.float32),
        pltpu.VMEM_SHARED((NPAD, D_IN), jnp.float32),
        pltpu.SemaphoreType.DMA,
        pltpu.SemaphoreType.DMA,
    ],
  )
  def _edge_fused(y_hbm, col_hbm, row_hbm, ea_hbm, out_hbm,
                  cidx0, cidx1, ridx0, ridx1, ea0, ea1, yg0, yg1, m0,
                  acc_sh, sem0, sem1):
    m1 = m0
    cid = lax.axis_index("c")
    sid = lax.axis_index("s")
    wid = sid * NC + cid
    base = wid * PER_W
    stripe = pl.ds(sid * ROWS_PER_TILE, ROWS_PER_TILE)

    bufs = ((cidx0, ridx0, ea0, yg0, m0, sem0),
            (cidx1, ridx1, ea1, yg1, m1, sem1))

    zed = jnp.zeros((L,), jnp.float32)

    def zrow(r, carry):
        for sj in range(D_IN // L):
            m0[r, pl.ds(sj * L, L)] = zed
        return carry

    lax.fori_loop(0, CHUNK, zrow, 0)

    for z in range(ROWS_PER_TILE // CHUNK):
        pltpu.sync_copy(m0, acc_sh.at[pl.ds(sid * ROWS_PER_TILE + z * CHUNK,
                                            CHUNK)])
    zrem = ROWS_PER_TILE - (ROWS_PER_TILE // CHUNK) * CHUNK
    if zrem:
        pltpu.sync_copy(
            m0.at[pl.ds(0, zrem)],
            acc_sh.at[pl.ds(sid * ROWS_PER_TILE
                            + (ROWS_PER_TILE // CHUNK) * CHUNK, zrem)])
    plsc.subcore_barrier()

    def edge_math(ea_v, yg_v, m_v):
        def body(e):
            av = ea_v[pl.ds(D_EDGE * e, L)]
            a0 = av[0]
            a1 = av[1]
            a2 = av[2]
            a3 = av[3]
            for q in range(D_H // L):
                v = (a0 * yg_v[e, pl.ds(q * L, L)]
                     + a1 * yg_v[e, pl.ds(32 + q * L, L)]
                     + a2 * yg_v[e, pl.ds(64 + q * L, L)]
                     + a3 * yg_v[e, pl.ds(96 + q * L, L)])
                s = 1.0 / (1.0 + jnp.exp(-v))
                m_v[e, pl.ds(q * L, L)] = v * s
        return body

    def issue(i, b):
        cidx, ridx, ea, yg, m, sem = bufs[b]
        off = base + i * CHUNK
        pltpu.sync_copy(col_hbm.at[pl.ds(off, CHUNK)], cidx)
        pltpu.sync_copy(row_hbm.at[pl.ds(off, CHUNK)], ridx)
        pltpu.sync_copy(ea_hbm.at[pl.ds(off * D_EDGE, CHUNK * D_EDGE)],
                        ea.at[pl.ds(0, CHUNK * D_EDGE)])
        return pltpu.async_copy(y_hbm.at[cidx], yg, sem)

    def crunch(b):
        cidx, ridx, ea, yg, m, sem = bufs[b]
        plsc.parallel_loop(0, CHUNK, 1, unroll=6)(edge_math(ea, yg, m))
        pltpu.sync_copy(m, acc_sh.at[ridx], add=True)

    issue(0, 0).wait()

    def pair_body(p, carry):
        d1 = issue(2 * p + 1, 1)
        crunch(0)
        d1.wait()
        d0 = issue(2 * p + 2, 0)
        crunch(1)
        d0.wait()
        return carry

    lax.fori_loop(0, NFULL // 2 - 1, pair_body, 0)
    dl = issue(NFULL - 1, 1)
    crunch(0)
    dl.wait()
    crunch(1)

    off = base + NFULL * CHUNK
    pltpu.sync_copy(col_hbm.at[pl.ds(off, TAIL)], cidx1.at[pl.ds(0, TAIL)])
    pltpu.sync_copy(row_hbm.at[pl.ds(off, TAIL)], ridx1.at[pl.ds(0, TAIL)])
    pltpu.sync_copy(ea_hbm.at[pl.ds(off * D_EDGE, TAIL * D_EDGE)],
                    ea1.at[pl.ds(0, TAIL * D_EDGE)])
    pltpu.async_copy(y_hbm.at[cidx1.at[pl.ds(0, TAIL)]],
                     yg1.at[pl.ds(0, TAIL)], sem1).wait()
    plsc.parallel_loop(0, TAIL, 1, unroll=4)(edge_math(ea1, yg1, m1))
    pltpu.sync_copy(m1.at[pl.ds(0, TAIL)],
                    acc_sh.at[ridx1.at[pl.ds(0, TAIL)]], add=True)

    plsc.subcore_barrier()
    pltpu.sync_copy(acc_sh.at[stripe], out_hbm.at[cid, stripe])

  return _edge_fused


BY = 2000


def _prep_body(x_ref, w1_ref, y_ref):
    y_ref[...] = jnp.dot(x_ref[...], w1_ref[...],
                         preferred_element_type=jnp.float32)


def _prep_stage(x, w1):
    return pl.pallas_call(
        _prep_body,
        grid=(N // BY,),
        in_specs=[
            pl.BlockSpec((BY, D_IN), lambda i: (i, 0)),
            pl.BlockSpec((D_IN, D_EDGE * D_H), lambda i: (0, 0)),
        ],
        out_specs=pl.BlockSpec((BY, D_EDGE * D_H), lambda i: (i, 0)),
        out_shape=jax.ShapeDtypeStruct((N, D_EDGE * D_H), jnp.float32),
    )(x, w1)


BN = 1000


def _update_body(x_ref, p0_ref, p1_ref, wl_ref, w2_ref, sel_ref, wl2_ref, out_ref):
    x = x_ref[...]
    agg = p0_ref[:, :D_H] + p1_ref[:, :D_H]
    agg = jnp.dot(agg, wl_ref[...], preferred_element_type=jnp.float32)
    t = jnp.dot(x, w2_ref[...], preferred_element_type=jnp.float32)
    a32 = jnp.tile(agg, (1, D_H))
    u = jnp.dot(t * a32, sel_ref[...], preferred_element_type=jnp.float32)
    u = u * jax.nn.sigmoid(u)
    out_ref[...] = x + jnp.dot(u, wl2_ref[...], preferred_element_type=jnp.float32)


def _update_stage(x, p0, p1, wl, w2, sel, wl2):
    return pl.pallas_call(
        _update_body,
        grid=(N // BN,),
        in_specs=[
            pl.BlockSpec((BN, D_IN), lambda i: (i, 0)),
            pl.BlockSpec((BN, D_IN), lambda i: (i, 0)),
            pl.BlockSpec((BN, D_IN), lambda i: (i, 0)),
            pl.BlockSpec((D_H, D_H), lambda i: (0, 0)),
            pl.BlockSpec((D_IN, D_H * D_H), lambda i: (0, 0)),
            pl.BlockSpec((D_H * D_H, D_H), lambda i: (0, 0)),
            pl.BlockSpec((D_H, D_IN), lambda i: (0, 0)),
        ],
        out_specs=pl.BlockSpec((BN, D_IN), lambda i: (i, 0)),
        out_shape=jax.ShapeDtypeStruct((N, D_IN), jnp.float32),
    )(x, p0, p1, wl, w2, sel, wl2)


def kernel(node_features, edge_index, edge_attr_e3nn, node_attr_scalar_raw,
           W_tp_msg, W_lin_msg, W_tp_upd, W_lin_upd):
    del node_attr_scalar_raw
    row = edge_index[0].astype(jnp.int32)
    col = edge_index[1].astype(jnp.int32)
    w1 = W_tp_msg.reshape(D_IN, D_EDGE * D_H)
    w2 = W_tp_upd.transpose(0, 2, 1).reshape(D_IN, D_H * D_H)
    sel = jnp.kron(jnp.eye(D_H, dtype=jnp.float32),
                   jnp.ones((D_H, 1), jnp.float32))
    ea1d = edge_attr_e3nn.reshape(-1)
    y = _prep_stage(node_features, w1)
    _edge_fused = _sc_kernels()
    partials = _edge_fused(y, col, row, ea1d)
    return _update_stage(node_features, partials[0, :N], partials[1, :N],
                         W_lin_msg, w2, sel, W_lin_upd)
